# baseline (speedup 1.0000x reference)
import functools

import jax
import jax.numpy as jnp
from jax import lax
from jax.experimental import pallas as pl
from jax.experimental.pallas import tpu as pltpu
from jax.experimental.pallas import tpu_sc as plsc

F32 = jnp.float32

NC, NS = 2, 16
NW = NC * NS
N = 10000
NPAD = 10112
E = 320000
EW = E // NW
CW = 100
NCH = EW // CW
RT = NPAD // NS
DH = 64
DOUT = 128
NBUF = 4
SDEPTH = 2

_mesh = plsc.VectorSubcoreMesh(
    core_axis_name="c", subcore_axis_name="s", num_cores=NC, num_subcores=NS
)
_sc_params = pltpu.CompilerParams(use_tc_tiling_on_sc=False)


def _fill(buf, rows, width, value):
    vec = jnp.full((16,), value, F32)

    @pl.loop(0, rows)
    def _(r):
        for k in range(width // 16):
            buf[r, pl.ds(k * 16, 16)] = vec


def _zero_spmem(buf, accum, r0, width):
    nfull = RT // CW
    rem = RT - nfull * CW

    @pl.loop(0, nfull)
    def _(i):
        pltpu.sync_copy(buf, accum.at[pl.ds(r0 + i * CW, CW)])

    pltpu.sync_copy(buf.at[pl.ds(0, rem)], accum.at[pl.ds(r0 + nfull * CW, rem)])


def _deg_body(dst_hbm, out_hbm, dst_v, ones_v, accum, gsem):
    cid = lax.axis_index("c")
    sid = lax.axis_index("s")
    wid = sid * NC + cid
    r0 = sid * RT

    pltpu.sync_copy(dst_hbm.at[wid], dst_v)
    _fill(ones_v, CW, 16, 0.0)
    _zero_spmem(ones_v, accum, r0, 16)
    _fill(ones_v, CW, 16, 1.0)
    plsc.subcore_barrier()

    def d_start(j, b):
        pltpu.async_copy(ones_v, accum.at[dst_v.at[j]], gsem.at[b], add=True)

    def d_wait(j, b):
        pltpu.make_async_copy(ones_v, accum.at[dst_v.at[j]], gsem.at[b]).wait()

    @pl.loop(0, NCH, step=4)
    def _(j0):
        for b in range(4):
            j = j0 + b

            @pl.when(j - 4 >= 0)
            def _():
                d_wait(j - 4, b)

            d_start(j, b)

    for b in range(4):
        d_wait(NCH - 4 + b, b)

    plsc.subcore_barrier()
    out_off = cid * NPAD + r0
    pltpu.sync_copy(accum.at[pl.ds(r0, RT)], out_hbm.at[pl.ds(out_off, RT)])


_deg_call = functools.partial(
    pl.kernel,
    out_type=jax.ShapeDtypeStruct((2 * NPAD, 16), F32),
    mesh=_mesh,
    scratch_types=[
        pltpu.VMEM((NCH, CW), jnp.int32),
        pltpu.VMEM((CW, 16), F32),
        pltpu.VMEM_SHARED((NPAD, 16), F32),
        pltpu.SemaphoreType.DMA((4,)),
    ],
    compiler_params=_sc_params,
)(_deg_body)


def _agg_body(src_hbm, dst_hbm, p_hbm, out_hbm,
              src_v, dst_v, rowbuf, accum, ptab, gsem, ssem):
    cid = lax.axis_index("c")
    sid = lax.axis_index("s")
    wid = sid * NC + cid
    r0 = sid * RT

    pltpu.sync_copy(src_hbm.at[wid], src_v)
    pltpu.sync_copy(dst_hbm.at[wid], dst_v)
    pltpu.sync_copy(p_hbm.at[pl.ds(r0, RT)], ptab.at[pl.ds(r0, RT)])

    @pl.when(cid == 0)
    def _():
        pltpu.sync_copy(p_hbm.at[pl.ds(r0, RT)], accum.at[pl.ds(r0, RT)])

    @pl.when(cid != 0)
    def _():
        _fill(rowbuf.at[0], CW, DH, 0.0)
        _zero_spmem(rowbuf.at[0], accum, r0, DH)

    plsc.subcore_barrier()

    def g_start(j, b):
        pltpu.async_copy(ptab.at[src_v.at[j]], rowbuf.at[b], gsem.at[b])

    def g_wait(j, b):
        pltpu.make_async_copy(ptab.at[src_v.at[j]], rowbuf.at[b], gsem.at[b]).wait()

    def s_start(j, b):
        pltpu.async_copy(rowbuf.at[b], accum.at[dst_v.at[j]], ssem.at[b], add=True)

    def s_wait(j, b):
        pltpu.make_async_copy(rowbuf.at[b], accum.at[dst_v.at[j]], ssem.at[b]).wait()

    GDEPTH = NBUF - SDEPTH
    for b in range(GDEPTH):
        g_start(b, b)

    @pl.loop(0, NCH, step=NBUF)
    def _(j0):
        for b in range(NBUF):
            j = j0 + b
            g_wait(j, b)
            s_start(j, b)

            @pl.when(j - SDEPTH >= 0)
            def _():
                s_wait(j - SDEPTH, (b + NBUF - SDEPTH) % NBUF)

            @pl.when(j + GDEPTH < NCH)
            def _():
                g_start(j + GDEPTH, (b + GDEPTH) % NBUF)

    for k in range(SDEPTH):
        j = NCH - SDEPTH + k
        s_wait(j, j % NBUF)

    plsc.subcore_barrier()
    out_off = cid * NPAD + r0
    pltpu.sync_copy(accum.at[pl.ds(r0, RT)], out_hbm.at[pl.ds(out_off, RT)])


_agg_call = functools.partial(
    pl.kernel,
    out_type=jax.ShapeDtypeStruct((2 * NPAD, DH), F32),
    mesh=_mesh,
    scratch_types=[
        pltpu.VMEM((NCH, CW), jnp.int32),
        pltpu.VMEM((NCH, CW), jnp.int32),
        pltpu.VMEM((NBUF, CW, DH), F32),
        pltpu.VMEM_SHARED((NPAD, DH), F32),
        pltpu.VMEM_SHARED((NPAD, DH), F32),
        pltpu.SemaphoreType.DMA((NBUF,)),
        pltpu.SemaphoreType.DMA((NBUF,)),
    ],
    compiler_params=_sc_params,
)(_agg_body)


_GRID = 8
_BR = NPAD // _GRID


def _tc_mm_body(x_ref, w_ref, p_ref):
    p_ref[...] = jnp.dot(x_ref[...], w_ref[...], preferred_element_type=F32)


def _tc_mm(x, w1):
    return pl.pallas_call(
        _tc_mm_body,
        grid=(10,),
        in_specs=[
            pl.BlockSpec((1000, 128), lambda j: (j, 0)),
            pl.BlockSpec((128, DH), lambda j: (0, 0)),
        ],
        out_specs=pl.BlockSpec((1000, DH), lambda j: (j, 0)),
        out_shape=jax.ShapeDtypeStruct((NPAD, DH), F32),
    )(x, w1)


def _tc_in_body(p_raw_ref, deg_ref, p_ref, dinv_ref):
    dv = lax.rsqrt(deg_ref[0, :, :1] + deg_ref[1, :, :1] + 1.0)
    p_ref[...] = p_raw_ref[...] * dv
    dinv_ref[...] = jnp.broadcast_to(dv, dinv_ref.shape)


def _tc_in(p_raw, deg2):
    return pl.pallas_call(
        _tc_in_body,
        grid=(10,),
        in_specs=[
            pl.BlockSpec((1000, DH), lambda j: (j, 0)),
            pl.BlockSpec((2, 1000, 16), lambda j: (0, j, 0)),
        ],
        out_specs=[
            pl.BlockSpec((1000, DH), lambda j: (j, 0)),
            pl.BlockSpec((1000, 16), lambda j: (j, 0)),
        ],
        out_shape=[
            jax.ShapeDtypeStruct((NPAD, DH), F32),
            jax.ShapeDtypeStruct((NPAD, 16), F32),
        ],
    )(p_raw, deg2)


def _tc_mid_body(s_ref, dinv_ref, b_ref, w_ref, out_ref):
    dv = dinv_ref[:, :1]
    a = (s_ref[0] + s_ref[1]) * dv
    h = jnp.maximum(a + b_ref[...], 0.0)
    out_ref[...] = jnp.dot(h, w_ref[...], preferred_element_type=F32) * dv


def _tc_mid(s2, dinv, b, w):
    return pl.pallas_call(
        _tc_mid_body,
        grid=(_GRID,),
        in_specs=[
            pl.BlockSpec((2, _BR, DH), lambda j: (0, j, 0)),
            pl.BlockSpec((_BR, 16), lambda j: (j, 0)),
            pl.BlockSpec((1, DH), lambda j: (0, 0)),
            pl.BlockSpec((DH, DH), lambda j: (0, 0)),
        ],
        out_specs=pl.BlockSpec((_BR, DH), lambda j: (j, 0)),
        out_shape=jax.ShapeDtypeStruct((NPAD, DH), F32),
    )(s2, dinv, b, w)


def _tc_out_body(s_ref, dinv_ref, w_ref, b_ref, out_ref):
    a = (s_ref[0] + s_ref[1]) * dinv_ref[:, :1]
    out_ref[...] = jnp.dot(a, w_ref[...], preferred_element_type=F32) + b_ref[...]


def _tc_out(s2, dinv, w3, b3):
    return pl.pallas_call(
        _tc_out_body,
        grid=(10,),
        in_specs=[
            pl.BlockSpec((2, 1000, DH), lambda j: (0, j, 0)),
            pl.BlockSpec((1000, 16), lambda j: (j, 0)),
            pl.BlockSpec((DH, DOUT), lambda j: (0, 0)),
            pl.BlockSpec((1, DOUT), lambda j: (0, 0)),
        ],
        out_specs=pl.BlockSpec((1000, DOUT), lambda j: (j, 0)),
        out_shape=jax.ShapeDtypeStruct((N, DOUT), F32),
    )(s2, dinv, w3, b3)


@jax.jit
def kernel(x, edge_index, W1, b1, W2, b2, W3, b3):
    ei = edge_index.astype(jnp.int32)
    src = ei[0].reshape(NW, NCH, CW)
    dst = ei[1].reshape(NW, NCH, CW)

    p1_raw = _tc_mm(x, W1)
    deg2 = _deg_call(dst).reshape(2, NPAD, 16)
    p1, dinv = _tc_in(p1_raw, deg2)
    s1 = _agg_call(src, dst, p1).reshape(2, NPAD, DH)
    p2 = _tc_mid(s1, dinv, b1.reshape(1, DH), W2)
    s2 = _agg_call(src, dst, p2).reshape(2, NPAD, DH)
    p3 = _tc_mid(s2, dinv, b2.reshape(1, DH), jnp.eye(DH, dtype=F32))
    s3 = _agg_call(src, dst, p3).reshape(2, NPAD, DH)
    return _tc_out(s3, dinv, W3, b3.reshape(1, DOUT))


# speedup vs baseline: 1.0059x; 1.0059x over previous
import functools

import jax
import jax.numpy as jnp
from jax import lax
from jax.experimental import pallas as pl
from jax.experimental.pallas import tpu as pltpu
from jax.experimental.pallas import tpu_sc as plsc

F32 = jnp.float32

NC, NS = 2, 16
NW = NC * NS
N = 10000
NPAD = 10112
E = 320000
EW = E // NW
CW = 100
NCH = EW // CW
RT = NPAD // NS
DH = 64
DOUT = 128
NBUF = 4
SDEPTH = 2

_mesh = plsc.VectorSubcoreMesh(
    core_axis_name="c", subcore_axis_name="s", num_cores=NC, num_subcores=NS
)
_sc_params = pltpu.CompilerParams(use_tc_tiling_on_sc=False)


def _fill(buf, rows, width, value):
    vec = jnp.full((16,), value, F32)

    @pl.loop(0, rows)
    def _(r):
        for k in range(width // 16):
            buf[r, pl.ds(k * 16, 16)] = vec


def _zero_spmem(buf, accum, r0, width):
    nfull = RT // CW
    rem = RT - nfull * CW

    @pl.loop(0, nfull)
    def _(i):
        pltpu.sync_copy(buf, accum.at[pl.ds(r0 + i * CW, CW)])

    pltpu.sync_copy(buf.at[pl.ds(0, rem)], accum.at[pl.ds(r0 + nfull * CW, rem)])


def _deg_body(dst_hbm, out_hbm, dst_v, ones_v, accum, gsem):
    cid = lax.axis_index("c")
    sid = lax.axis_index("s")
    wid = sid * NC + cid
    r0 = sid * RT

    pltpu.sync_copy(dst_hbm.at[wid], dst_v)
    _fill(ones_v, CW, 16, 0.0)
    _zero_spmem(ones_v, accum, r0, 16)
    _fill(ones_v, CW, 16, 1.0)
    plsc.subcore_barrier()

    def d_start(j, b):
        pltpu.async_copy(ones_v, accum.at[dst_v.at[j]], gsem.at[b], add=True)

    def d_wait(j, b):
        pltpu.make_async_copy(ones_v, accum.at[dst_v.at[j]], gsem.at[b]).wait()

    @pl.loop(0, NCH, step=4)
    def _(j0):
        for b in range(4):
            j = j0 + b

            @pl.when(j - 4 >= 0)
            def _():
                d_wait(j - 4, b)

            d_start(j, b)

    for b in range(4):
        d_wait(NCH - 4 + b, b)

    plsc.subcore_barrier()
    out_off = cid * NPAD + r0
    pltpu.sync_copy(accum.at[pl.ds(r0, RT)], out_hbm.at[pl.ds(out_off, RT)])


_deg_call = functools.partial(
    pl.kernel,
    out_type=jax.ShapeDtypeStruct((2 * NPAD, 16), F32),
    mesh=_mesh,
    scratch_types=[
        pltpu.VMEM((NCH, CW), jnp.int32),
        pltpu.VMEM((CW, 16), F32),
        pltpu.VMEM_SHARED((NPAD, 16), F32),
        pltpu.SemaphoreType.DMA((4,)),
    ],
    compiler_params=_sc_params,
)(_deg_body)


def _agg_body(src_hbm, dst_hbm, p_hbm, out_hbm,
              src_v, dst_v, rowbuf, accum, ptab, gsem, ssem):
    cid = lax.axis_index("c")
    sid = lax.axis_index("s")
    wid = sid * NC + cid
    r0 = sid * RT

    pltpu.sync_copy(src_hbm.at[wid], src_v)
    pltpu.sync_copy(dst_hbm.at[wid], dst_v)
    pltpu.sync_copy(p_hbm.at[pl.ds(r0, RT)], ptab.at[pl.ds(r0, RT)])

    @pl.when(cid == 0)
    def _():
        pltpu.sync_copy(p_hbm.at[pl.ds(r0, RT)], accum.at[pl.ds(r0, RT)])

    @pl.when(cid != 0)
    def _():
        _fill(rowbuf.at[0], CW, DH, 0.0)
        _zero_spmem(rowbuf.at[0], accum, r0, DH)

    plsc.subcore_barrier()

    def g_start(j, b):
        pltpu.async_copy(ptab.at[src_v.at[j]], rowbuf.at[b], gsem.at[b])

    def g_wait(j, b):
        pltpu.make_async_copy(ptab.at[src_v.at[j]], rowbuf.at[b], gsem.at[b]).wait()

    def s_start(j, b):
        pltpu.async_copy(rowbuf.at[b], accum.at[dst_v.at[j]], ssem.at[b], add=True)

    def s_wait(j, b):
        pltpu.make_async_copy(rowbuf.at[b], accum.at[dst_v.at[j]], ssem.at[b]).wait()

    GDEPTH = NBUF - SDEPTH
    for b in range(GDEPTH):
        g_start(b, b)

    @pl.loop(0, NCH, step=NBUF)
    def _(j0):
        for b in range(NBUF):
            j = j0 + b
            g_wait(j, b)
            s_start(j, b)

            @pl.when(j - SDEPTH >= 0)
            def _():
                s_wait(j - SDEPTH, (b + NBUF - SDEPTH) % NBUF)

            @pl.when(j + GDEPTH < NCH)
            def _():
                g_start(j + GDEPTH, (b + GDEPTH) % NBUF)

    for k in range(SDEPTH):
        j = NCH - SDEPTH + k
        s_wait(j, j % NBUF)

    plsc.subcore_barrier()
    out_off = cid * NPAD + r0
    pltpu.sync_copy(accum.at[pl.ds(r0, RT)], out_hbm.at[pl.ds(out_off, RT)])


_agg_call = functools.partial(
    pl.kernel,
    out_type=jax.ShapeDtypeStruct((2 * NPAD, DH), F32),
    mesh=_mesh,
    scratch_types=[
        pltpu.VMEM((NCH, CW), jnp.int32),
        pltpu.VMEM((NCH, CW), jnp.int32),
        pltpu.VMEM((NBUF, CW, DH), F32),
        pltpu.VMEM_SHARED((NPAD, DH), F32),
        pltpu.VMEM_SHARED((NPAD, DH), F32),
        pltpu.SemaphoreType.DMA((NBUF,)),
        pltpu.SemaphoreType.DMA((NBUF,)),
    ],
    compiler_params=_sc_params,
)(_agg_body)


_GRID = 8
_BR = NPAD // _GRID


def _tc_in_body(x_ref, w_ref, deg_ref, p_ref, dinv_ref):
    dv = lax.rsqrt(deg_ref[0, :, :1] + deg_ref[1, :, :1] + 1.0)
    p = jnp.dot(x_ref[...], w_ref[...], preferred_element_type=F32)
    p_ref[...] = p * dv
    dinv_ref[...] = jnp.broadcast_to(dv, dinv_ref.shape)


def _tc_in(x, w1, deg2):
    return pl.pallas_call(
        _tc_in_body,
        grid=(10,),
        in_specs=[
            pl.BlockSpec((1000, 128), lambda j: (j, 0)),
            pl.BlockSpec((128, DH), lambda j: (0, 0)),
            pl.BlockSpec((2, 1000, 16), lambda j: (0, j, 0)),
        ],
        out_specs=[
            pl.BlockSpec((1000, DH), lambda j: (j, 0)),
            pl.BlockSpec((1000, 16), lambda j: (j, 0)),
        ],
        out_shape=[
            jax.ShapeDtypeStruct((NPAD, DH), F32),
            jax.ShapeDtypeStruct((NPAD, 16), F32),
        ],
    )(x, w1, deg2)


def _tc_mid_body(s_ref, dinv_ref, b_ref, w_ref, out_ref):
    dv = dinv_ref[:, :1]
    a = (s_ref[0] + s_ref[1]) * dv
    h = jnp.maximum(a + b_ref[...], 0.0)
    out_ref[...] = jnp.dot(h, w_ref[...], preferred_element_type=F32) * dv


def _tc_mid(s2, dinv, b, w):
    return pl.pallas_call(
        _tc_mid_body,
        grid=(_GRID,),
        in_specs=[
            pl.BlockSpec((2, _BR, DH), lambda j: (0, j, 0)),
            pl.BlockSpec((_BR, 16), lambda j: (j, 0)),
            pl.BlockSpec((1, DH), lambda j: (0, 0)),
            pl.BlockSpec((DH, DH), lambda j: (0, 0)),
        ],
        out_specs=pl.BlockSpec((_BR, DH), lambda j: (j, 0)),
        out_shape=jax.ShapeDtypeStruct((NPAD, DH), F32),
    )(s2, dinv, b, w)


def _tc_out_body(s_ref, dinv_ref, w_ref, b_ref, out_ref):
    a = (s_ref[0] + s_ref[1]) * dinv_ref[:, :1]
    out_ref[...] = jnp.dot(a, w_ref[...], preferred_element_type=F32) + b_ref[...]


def _tc_out(s2, dinv, w3, b3):
    return pl.pallas_call(
        _tc_out_body,
        grid=(10,),
        in_specs=[
            pl.BlockSpec((2, 1000, DH), lambda j: (0, j, 0)),
            pl.BlockSpec((1000, 16), lambda j: (j, 0)),
            pl.BlockSpec((DH, DOUT), lambda j: (0, 0)),
            pl.BlockSpec((1, DOUT), lambda j: (0, 0)),
        ],
        out_specs=pl.BlockSpec((1000, DOUT), lambda j: (j, 0)),
        out_shape=jax.ShapeDtypeStruct((N, DOUT), F32),
    )(s2, dinv, w3, b3)


@jax.jit
def kernel(x, edge_index, W1, b1, W2, b2, W3, b3):
    ei = edge_index.astype(jnp.int32)
    src = ei[0].reshape(NW, NCH, CW)
    dst = ei[1].reshape(NW, NCH, CW)

    deg2 = _deg_call(dst).reshape(2, NPAD, 16)
    p1, dinv = _tc_in(x, W1, deg2)
    s1 = _agg_call(src, dst, p1).reshape(2, NPAD, DH)
    p2 = _tc_mid(s1, dinv, b1.reshape(1, DH), W2)
    s2 = _agg_call(src, dst, p2).reshape(2, NPAD, DH)
    p3 = _tc_mid(s2, dinv, b2.reshape(1, DH), jnp.eye(DH, dtype=F32))
    s3 = _agg_call(src, dst, p3).reshape(2, NPAD, DH)
    return _tc_out(s3, dinv, W3, b3.reshape(1, DOUT))


# speedup vs baseline: 1.0349x; 1.0288x over previous
import functools

import jax
import jax.numpy as jnp
from jax import lax
from jax.experimental import pallas as pl
from jax.experimental.pallas import tpu as pltpu
from jax.experimental.pallas import tpu_sc as plsc

F32 = jnp.float32

NC, NS = 2, 16
NW = NC * NS
N = 10000
NPAD = 10112
E = 320000
EW = E // NW
CW = 100
NCH = EW // CW
RT = NPAD // NS
DH = 64
DOUT = 128
NBUF = 4
SDEPTH = 2

_mesh = plsc.VectorSubcoreMesh(
    core_axis_name="c", subcore_axis_name="s", num_cores=NC, num_subcores=NS
)
_sc_params = pltpu.CompilerParams(use_tc_tiling_on_sc=False)


def _fill(buf, rows, width, value):
    vec = jnp.full((16,), value, F32)

    @pl.loop(0, rows)
    def _(r):
        for k in range(width // 16):
            buf[r, pl.ds(k * 16, 16)] = vec


def _zero_spmem(buf, accum, r0, width):
    nfull = RT // CW
    rem = RT - nfull * CW

    @pl.loop(0, nfull)
    def _(i):
        pltpu.sync_copy(buf, accum.at[pl.ds(r0 + i * CW, CW)])

    pltpu.sync_copy(buf.at[pl.ds(0, rem)], accum.at[pl.ds(r0 + nfull * CW, rem)])


def _deg_body(dst_hbm, out_hbm, dst_v, ones_v, accum, gsem):
    cid = lax.axis_index("c")
    sid = lax.axis_index("s")
    wid = sid * NC + cid
    r0 = sid * RT

    pltpu.sync_copy(dst_hbm.at[wid], dst_v)
    _fill(ones_v, CW, 16, 0.0)
    _zero_spmem(ones_v, accum, r0, 16)
    _fill(ones_v, CW, 16, 1.0)
    plsc.subcore_barrier()

    def d_start(j, b):
        pltpu.async_copy(ones_v, accum.at[dst_v.at[j]], gsem.at[b], add=True)

    def d_wait(j, b):
        pltpu.make_async_copy(ones_v, accum.at[dst_v.at[j]], gsem.at[b]).wait()

    @pl.loop(0, NCH, step=4)
    def _(j0):
        for b in range(4):
            j = j0 + b

            @pl.when(j - 4 >= 0)
            def _():
                d_wait(j - 4, b)

            d_start(j, b)

    for b in range(4):
        d_wait(NCH - 4 + b, b)

    plsc.subcore_barrier()
    out_off = cid * NPAD + r0
    pltpu.sync_copy(accum.at[pl.ds(r0, RT)], out_hbm.at[pl.ds(out_off, RT)])


_deg_call = functools.partial(
    pl.kernel,
    out_type=jax.ShapeDtypeStruct((2 * NPAD, 16), F32),
    mesh=_mesh,
    scratch_types=[
        pltpu.VMEM((NCH, CW), jnp.int32),
        pltpu.VMEM((CW, 16), F32),
        pltpu.VMEM_SHARED((NPAD, 16), F32),
        pltpu.SemaphoreType.DMA((4,)),
    ],
    compiler_params=_sc_params,
)(_deg_body)


def _agg_body(src_hbm, dst_hbm, p_hbm, out_hbm,
              src_v, dst_v, rowbuf, accum, ptab, gsem, ssem):
    cid = lax.axis_index("c")
    sid = lax.axis_index("s")
    wid = sid * NC + cid
    r0 = sid * RT

    pltpu.sync_copy(src_hbm.at[wid], src_v)
    pltpu.sync_copy(dst_hbm.at[wid], dst_v)
    pltpu.sync_copy(p_hbm.at[pl.ds(r0, RT)], ptab.at[pl.ds(r0, RT)])

    @pl.when(cid == 0)
    def _():
        pltpu.sync_copy(p_hbm.at[pl.ds(r0, RT)], accum.at[pl.ds(r0, RT)])

    @pl.when(cid != 0)
    def _():
        _fill(rowbuf.at[0], CW, DH, 0.0)
        _zero_spmem(rowbuf.at[0], accum, r0, DH)

    plsc.subcore_barrier()

    def g_start(j, b):
        pltpu.async_copy(ptab.at[src_v.at[j]], rowbuf.at[b], gsem.at[b])

    def g_wait(j, b):
        pltpu.make_async_copy(ptab.at[src_v.at[j]], rowbuf.at[b], gsem.at[b]).wait()

    def s_start(j, b):
        pltpu.async_copy(rowbuf.at[b], accum.at[dst_v.at[j]], ssem.at[b], add=True)

    def s_wait(j, b):
        pltpu.make_async_copy(rowbuf.at[b], accum.at[dst_v.at[j]], ssem.at[b]).wait()

    GDEPTH = NBUF - SDEPTH
    for b in range(GDEPTH):
        g_start(b, b)

    @pl.loop(0, NCH, step=NBUF)
    def _(j0):
        for b in range(NBUF):
            j = j0 + b
            g_wait(j, b)
            s_start(j, b)

            @pl.when(j - SDEPTH >= 0)
            def _():
                s_wait(j - SDEPTH, (b + NBUF - SDEPTH) % NBUF)

            @pl.when(j + GDEPTH < NCH)
            def _():
                g_start(j + GDEPTH, (b + GDEPTH) % NBUF)

    for k in range(SDEPTH):
        j = NCH - SDEPTH + k
        s_wait(j, j % NBUF)

    plsc.subcore_barrier()
    out_off = cid * NPAD + r0
    pltpu.sync_copy(accum.at[pl.ds(r0, RT)], out_hbm.at[pl.ds(out_off, RT)])


_agg_call = functools.partial(
    pl.kernel,
    out_type=jax.ShapeDtypeStruct((2 * NPAD, DH), F32),
    mesh=_mesh,
    scratch_types=[
        pltpu.VMEM((NCH, CW), jnp.int32),
        pltpu.VMEM((NCH, CW), jnp.int32),
        pltpu.VMEM((NBUF, CW, DH), F32),
        pltpu.VMEM_SHARED((NPAD, DH), F32),
        pltpu.VMEM_SHARED((NPAD, DH), F32),
        pltpu.SemaphoreType.DMA((NBUF,)),
        pltpu.SemaphoreType.DMA((NBUF,)),
    ],
    compiler_params=_sc_params,
)(_agg_body)


_GRID = 4
_BR = NPAD // _GRID


def _tc_in_body(x_ref, w_ref, deg_ref, p_ref, dinv_ref):
    dv = lax.rsqrt(deg_ref[0, :, :1] + deg_ref[1, :, :1] + 1.0)
    p = jnp.dot(x_ref[...], w_ref[...], preferred_element_type=F32)
    p_ref[...] = p * dv
    dinv_ref[...] = jnp.broadcast_to(dv, dinv_ref.shape)


def _tc_in(x, w1, deg2):
    return pl.pallas_call(
        _tc_in_body,
        grid=(5,),
        in_specs=[
            pl.BlockSpec((2000, 128), lambda j: (j, 0)),
            pl.BlockSpec((128, DH), lambda j: (0, 0)),
            pl.BlockSpec((2, 2000, 16), lambda j: (0, j, 0)),
        ],
        out_specs=[
            pl.BlockSpec((2000, DH), lambda j: (j, 0)),
            pl.BlockSpec((2000, 16), lambda j: (j, 0)),
        ],
        out_shape=[
            jax.ShapeDtypeStruct((NPAD, DH), F32),
            jax.ShapeDtypeStruct((NPAD, 16), F32),
        ],
    )(x, w1, deg2)


def _tc_mid_body(s_ref, dinv_ref, b_ref, w_ref, out_ref):
    dv = dinv_ref[:, :1]
    a = (s_ref[0] + s_ref[1]) * dv
    h = jnp.maximum(a + b_ref[...], 0.0)
    out_ref[...] = jnp.dot(h, w_ref[...], preferred_element_type=F32) * dv


def _tc_mid(s2, dinv, b, w):
    return pl.pallas_call(
        _tc_mid_body,
        grid=(_GRID,),
        in_specs=[
            pl.BlockSpec((2, _BR, DH), lambda j: (0, j, 0)),
            pl.BlockSpec((_BR, 16), lambda j: (j, 0)),
            pl.BlockSpec((1, DH), lambda j: (0, 0)),
            pl.BlockSpec((DH, DH), lambda j: (0, 0)),
        ],
        out_specs=pl.BlockSpec((_BR, DH), lambda j: (j, 0)),
        out_shape=jax.ShapeDtypeStruct((NPAD, DH), F32),
    )(s2, dinv, b, w)


def _tc_out_body(s_ref, dinv_ref, w_ref, b_ref, out_ref):
    a = (s_ref[0] + s_ref[1]) * dinv_ref[:, :1]
    out_ref[...] = jnp.dot(a, w_ref[...], preferred_element_type=F32) + b_ref[...]


def _tc_out(s2, dinv, w3, b3):
    return pl.pallas_call(
        _tc_out_body,
        grid=(5,),
        in_specs=[
            pl.BlockSpec((2, 2000, DH), lambda j: (0, j, 0)),
            pl.BlockSpec((2000, 16), lambda j: (j, 0)),
            pl.BlockSpec((DH, DOUT), lambda j: (0, 0)),
            pl.BlockSpec((1, DOUT), lambda j: (0, 0)),
        ],
        out_specs=pl.BlockSpec((2000, DOUT), lambda j: (j, 0)),
        out_shape=jax.ShapeDtypeStruct((N, DOUT), F32),
    )(s2, dinv, w3, b3)


@jax.jit
def kernel(x, edge_index, W1, b1, W2, b2, W3, b3):
    ei = edge_index.astype(jnp.int32)
    src = ei[0].reshape(NW, NCH, CW)
    dst = ei[1].reshape(NW, NCH, CW)

    deg2 = _deg_call(dst).reshape(2, NPAD, 16)
    p1, dinv = _tc_in(x, W1, deg2)
    s1 = _agg_call(src, dst, p1).reshape(2, NPAD, DH)
    p2 = _tc_mid(s1, dinv, b1.reshape(1, DH), W2)
    s2 = _agg_call(src, dst, p2).reshape(2, NPAD, DH)
    p3 = _tc_mid(s2, dinv, b2.reshape(1, DH), jnp.eye(DH, dtype=F32))
    s3 = _agg_call(src, dst, p3).reshape(2, NPAD, DH)
    return _tc_out(s3, dinv, W3, b3.reshape(1, DOUT))


# speedup vs baseline: 1.0506x; 1.0152x over previous
import functools

import jax
import jax.numpy as jnp
from jax import lax
from jax.experimental import pallas as pl
from jax.experimental.pallas import tpu as pltpu
from jax.experimental.pallas import tpu_sc as plsc

F32 = jnp.float32

NC, NS = 2, 16
NW = NC * NS
N = 10000
NPAD = 10112
E = 320000
EW = E // NW
CW = 100
NCH = EW // CW
RT = NPAD // NS
DH = 64
DOUT = 128
NBUF = 4
SDEPTH = 2

_mesh = plsc.VectorSubcoreMesh(
    core_axis_name="c", subcore_axis_name="s", num_cores=NC, num_subcores=NS
)
_sc_params = pltpu.CompilerParams(use_tc_tiling_on_sc=False)


def _fill(buf, rows, width, value):
    vec = jnp.full((16,), value, F32)

    @pl.loop(0, rows)
    def _(r):
        for k in range(width // 16):
            buf[r, pl.ds(k * 16, 16)] = vec


def _zero_spmem(buf, accum, r0, width):
    nfull = RT // CW
    rem = RT - nfull * CW

    @pl.loop(0, nfull)
    def _(i):
        pltpu.sync_copy(buf, accum.at[pl.ds(r0 + i * CW, CW)])

    pltpu.sync_copy(buf.at[pl.ds(0, rem)], accum.at[pl.ds(r0 + nfull * CW, rem)])


def _deg_body(dst_hbm, out_hbm, dst_v, ones_v, accum, gsem):
    cid = lax.axis_index("c")
    sid = lax.axis_index("s")
    wid = sid * NC + cid
    r0 = sid * RT

    pltpu.sync_copy(dst_hbm.at[wid], dst_v)
    _fill(ones_v, CW, 16, 0.0)
    _zero_spmem(ones_v, accum, r0, 16)
    _fill(ones_v, CW, 16, 1.0)
    plsc.subcore_barrier()

    def d_start(j, b):
        pltpu.async_copy(ones_v, accum.at[dst_v.at[j]], gsem.at[b], add=True)

    def d_wait(j, b):
        pltpu.make_async_copy(ones_v, accum.at[dst_v.at[j]], gsem.at[b]).wait()

    @pl.loop(0, NCH, step=4)
    def _(j0):
        for b in range(4):
            j = j0 + b

            @pl.when(j - 4 >= 0)
            def _():
                d_wait(j - 4, b)

            d_start(j, b)

    for b in range(4):
        d_wait(NCH - 4 + b, b)

    plsc.subcore_barrier()
    out_off = cid * NPAD + r0
    pltpu.sync_copy(accum.at[pl.ds(r0, RT)], out_hbm.at[pl.ds(out_off, RT)])


_deg_call = functools.partial(
    pl.kernel,
    out_type=jax.ShapeDtypeStruct((2 * NPAD, 16), F32),
    mesh=_mesh,
    scratch_types=[
        pltpu.VMEM((NCH, CW), jnp.int32),
        pltpu.VMEM((CW, 16), F32),
        pltpu.VMEM_SHARED((NPAD, 16), F32),
        pltpu.SemaphoreType.DMA((4,)),
    ],
    compiler_params=_sc_params,
)(_deg_body)


def _agg_body(src_hbm, dst_hbm, p_hbm, out_hbm,
              src_v, dst_v, rowbuf, accum, ptab, gsem, ssem):
    cid = lax.axis_index("c")
    sid = lax.axis_index("s")
    wid = sid * NC + cid
    r0 = sid * RT

    pltpu.sync_copy(src_hbm.at[wid], src_v)
    pltpu.sync_copy(dst_hbm.at[wid], dst_v)
    pltpu.sync_copy(p_hbm.at[pl.ds(r0, RT)], ptab.at[pl.ds(r0, RT)])

    @pl.when(cid == 0)
    def _():
        pltpu.sync_copy(p_hbm.at[pl.ds(r0, RT)], accum.at[pl.ds(r0, RT)])

    @pl.when(cid != 0)
    def _():
        _fill(rowbuf.at[0], CW, DH, 0.0)
        _zero_spmem(rowbuf.at[0], accum, r0, DH)

    plsc.subcore_barrier()

    def g_start(j, b):
        pltpu.async_copy(ptab.at[src_v.at[j]], rowbuf.at[b], gsem.at[b])

    def g_wait(j, b):
        pltpu.make_async_copy(ptab.at[src_v.at[j]], rowbuf.at[b], gsem.at[b]).wait()

    def s_start(j, b):
        pltpu.async_copy(rowbuf.at[b], accum.at[dst_v.at[j]], ssem.at[b], add=True)

    def s_wait(j, b):
        pltpu.make_async_copy(rowbuf.at[b], accum.at[dst_v.at[j]], ssem.at[b]).wait()

    GDEPTH = NBUF - SDEPTH
    for b in range(GDEPTH):
        g_start(b, b)

    @pl.loop(0, NCH, step=NBUF)
    def _(j0):
        for b in range(NBUF):
            j = j0 + b
            g_wait(j, b)
            s_start(j, b)

            @pl.when(j - SDEPTH >= 0)
            def _():
                s_wait(j - SDEPTH, (b + NBUF - SDEPTH) % NBUF)

            @pl.when(j + GDEPTH < NCH)
            def _():
                g_start(j + GDEPTH, (b + GDEPTH) % NBUF)

    for k in range(SDEPTH):
        j = NCH - SDEPTH + k
        s_wait(j, j % NBUF)

    plsc.subcore_barrier()
    out_off = cid * NPAD + r0
    pltpu.sync_copy(accum.at[pl.ds(r0, RT)], out_hbm.at[pl.ds(out_off, RT)])


_agg_call = functools.partial(
    pl.kernel,
    out_type=jax.ShapeDtypeStruct((2 * NPAD, DH), F32),
    mesh=_mesh,
    scratch_types=[
        pltpu.VMEM((NCH, CW), jnp.int32),
        pltpu.VMEM((NCH, CW), jnp.int32),
        pltpu.VMEM((NBUF, CW, DH), F32),
        pltpu.VMEM_SHARED((NPAD, DH), F32),
        pltpu.VMEM_SHARED((NPAD, DH), F32),
        pltpu.SemaphoreType.DMA((NBUF,)),
        pltpu.SemaphoreType.DMA((NBUF,)),
    ],
    compiler_params=_sc_params,
)(_agg_body)


_GRID = 2
_BR = NPAD // _GRID


def _tc_in_body(x_ref, w_ref, deg_ref, p_ref, dinv_ref):
    dv = lax.rsqrt(deg_ref[0, :, :1] + deg_ref[1, :, :1] + 1.0)
    p = jnp.dot(x_ref[...], w_ref[...], preferred_element_type=F32)
    p_ref[...] = p * dv
    dinv_ref[...] = jnp.broadcast_to(dv, dinv_ref.shape)


def _tc_in(x, w1, deg2):
    return pl.pallas_call(
        _tc_in_body,
        grid=(2,),
        in_specs=[
            pl.BlockSpec((5000, 128), lambda j: (j, 0)),
            pl.BlockSpec((128, DH), lambda j: (0, 0)),
            pl.BlockSpec((2, 5000, 16), lambda j: (0, j, 0)),
        ],
        out_specs=[
            pl.BlockSpec((5000, DH), lambda j: (j, 0)),
            pl.BlockSpec((5000, 16), lambda j: (j, 0)),
        ],
        out_shape=[
            jax.ShapeDtypeStruct((NPAD, DH), F32),
            jax.ShapeDtypeStruct((NPAD, 16), F32),
        ],
    )(x, w1, deg2)


def _tc_mid_body(s_ref, dinv_ref, b_ref, w_ref, out_ref):
    dv = dinv_ref[:, :1]
    a = (s_ref[0] + s_ref[1]) * dv
    h = jnp.maximum(a + b_ref[...], 0.0)
    out_ref[...] = jnp.dot(h, w_ref[...], preferred_element_type=F32) * dv


def _tc_mid(s2, dinv, b, w):
    return pl.pallas_call(
        _tc_mid_body,
        grid=(_GRID,),
        in_specs=[
            pl.BlockSpec((2, _BR, DH), lambda j: (0, j, 0)),
            pl.BlockSpec((_BR, 16), lambda j: (j, 0)),
            pl.BlockSpec((1, DH), lambda j: (0, 0)),
            pl.BlockSpec((DH, DH), lambda j: (0, 0)),
        ],
        out_specs=pl.BlockSpec((_BR, DH), lambda j: (j, 0)),
        out_shape=jax.ShapeDtypeStruct((NPAD, DH), F32),
    )(s2, dinv, b, w)


def _tc_out_body(s_ref, dinv_ref, w_ref, b_ref, out_ref):
    a = (s_ref[0] + s_ref[1]) * dinv_ref[:, :1]
    out_ref[...] = jnp.dot(a, w_ref[...], preferred_element_type=F32) + b_ref[...]


def _tc_out(s2, dinv, w3, b3):
    return pl.pallas_call(
        _tc_out_body,
        grid=(2,),
        in_specs=[
            pl.BlockSpec((2, 5000, DH), lambda j: (0, j, 0)),
            pl.BlockSpec((5000, 16), lambda j: (j, 0)),
            pl.BlockSpec((DH, DOUT), lambda j: (0, 0)),
            pl.BlockSpec((1, DOUT), lambda j: (0, 0)),
        ],
        out_specs=pl.BlockSpec((5000, DOUT), lambda j: (j, 0)),
        out_shape=jax.ShapeDtypeStruct((N, DOUT), F32),
    )(s2, dinv, w3, b3)


@jax.jit
def kernel(x, edge_index, W1, b1, W2, b2, W3, b3):
    ei = edge_index.astype(jnp.int32)
    src = ei[0].reshape(NW, NCH, CW)
    dst = ei[1].reshape(NW, NCH, CW)

    deg2 = _deg_call(dst).reshape(2, NPAD, 16)
    p1, dinv = _tc_in(x, W1, deg2)
    s1 = _agg_call(src, dst, p1).reshape(2, NPAD, DH)
    p2 = _tc_mid(s1, dinv, b1.reshape(1, DH), W2)
    s2 = _agg_call(src, dst, p2).reshape(2, NPAD, DH)
    p3 = _tc_mid(s2, dinv, b2.reshape(1, DH), jnp.eye(DH, dtype=F32))
    s3 = _agg_call(src, dst, p3).reshape(2, NPAD, DH)
    return _tc_out(s3, dinv, W3, b3.reshape(1, DOUT))
